# Initial kernel scaffold; baseline (speedup 1.0000x reference)
import jax
import jax.numpy as jnp
from jax.experimental import pallas as pl


def kernel(t, eye):
    raise NotImplementedError("write your pallas kernel here")


# speedup vs baseline: 1.3148x; 1.3148x over previous
import jax
import jax.numpy as jnp
from jax.experimental import pallas as pl

_N, _J = 1024, 50
_C = 1000
_BI = 32


def _onehot_block(t_ref, out_ref):
    tv = t_ref[...]
    cls = jax.lax.broadcasted_iota(jnp.int32, (_BI, _C, _J), 1)
    out_ref[...] = (cls == tv[:, None, :]).astype(jnp.float32)


def kernel(t, eye):
    del eye
    t32 = t.astype(jnp.int32)
    grid = _N // _BI
    return pl.pallas_call(
        _onehot_block,
        grid=(grid,),
        in_specs=[pl.BlockSpec((_BI, _J), lambda i: (i, 0))],
        out_specs=pl.BlockSpec((_BI, _C, _J), lambda i: (i, 0, 0)),
        out_shape=jax.ShapeDtypeStruct((_N, _C, _J), jnp.float32),
    )(t32)


# speedup vs baseline: 1.3156x; 1.0006x over previous
import jax
import jax.numpy as jnp
from jax.experimental import pallas as pl
from jax.experimental.pallas import tpu as pltpu

_N, _J = 1024, 50
_C = 1000
_BI = 32


def _onehot_block(t_ref, out_ref):
    tv = t_ref[...]
    cls = jax.lax.broadcasted_iota(jnp.int32, (_BI, _C, _J), 1)
    out_ref[...] = (cls == tv[:, None, :]).astype(jnp.float32)


def kernel(t, eye):
    del eye
    t32 = t.astype(jnp.int32)
    grid = _N // _BI
    return pl.pallas_call(
        _onehot_block,
        grid=(grid,),
        in_specs=[pl.BlockSpec((_BI, _J), lambda i: (i, 0))],
        out_specs=pl.BlockSpec((_BI, _C, _J), lambda i: (i, 0, 0)),
        out_shape=jax.ShapeDtypeStruct((_N, _C, _J), jnp.float32),
        compiler_params=pltpu.CompilerParams(
            dimension_semantics=("parallel",),
        ),
    )(t32)


# speedup vs baseline: 10.7935x; 8.2041x over previous
import jax
import jax.numpy as jnp
from jax.experimental import pallas as pl
from jax.experimental.pallas import tpu as pltpu

_N, _J = 1024, 50
_C = 1000
_BJ = 2


def _onehot_block(tT_ref, out_ref):
    tv = tT_ref[0]
    cls = jax.lax.broadcasted_iota(jnp.int32, (_BJ, _C, _N), 1)
    out_ref[...] = (cls == tv[:, None, :]).astype(jnp.float32)


def kernel(t, eye):
    del eye
    tT = t.astype(jnp.int32).T.reshape(_J // _BJ, _BJ, _N)
    w = pl.pallas_call(
        _onehot_block,
        grid=(_J // _BJ,),
        in_specs=[pl.BlockSpec((1, _BJ, _N), lambda j: (j, 0, 0))],
        out_specs=pl.BlockSpec((_BJ, _C, _N), lambda j: (j, 0, 0)),
        out_shape=jax.ShapeDtypeStruct((_J, _C, _N), jnp.float32),
        compiler_params=pltpu.CompilerParams(
            dimension_semantics=("arbitrary",),
        ),
    )(tT)
    return w.transpose(2, 1, 0)
